# baseline (speedup 1.0000x reference)
import functools

import jax
import jax.numpy as jnp
from jax import lax
from jax.experimental import pallas as pl
from jax.experimental.pallas import tpu as pltpu
from jax.experimental.pallas import tpu_sc as plsc

N_TILES = 32
SUBCORES = 16
CHUNK = 48
LANES = 16


def _sc_agg_body(idx3_hbm, nodes_hbm, eemb_hbm, z_rows_hbm,
                 z_cnt_hbm, ones_hbm, sum_out, cnt_out,
                 raw_a, raw_b, idx_a, idx_b, buf_a, buf_b, buf_c, buf_d,
                 ones_v, acc_sum, acc_cnt, sem_a, sem_b, sem_c, sem_d):
    n_chunks_total = idx3_hbm.shape[0]
    n_pad, d = acc_sum.shape
    rows_per_tile = n_pad // SUBCORES
    cid = lax.axis_index("c")
    sid = lax.axis_index("s")
    tile = cid * SUBCORES + sid

    pltpu.sync_copy(ones_hbm, ones_v)
    for s in range(SUBCORES):
        @pl.when(sid == s)
        def _():
            pltpu.sync_copy(z_rows_hbm, acc_sum.at[pl.ds(s * rows_per_tile,
                                                         rows_per_tile)])
            pltpu.sync_copy(z_cnt_hbm, acc_cnt.at[pl.ds(s * rows_per_tile,
                                                        rows_per_tile)])
    plsc.subcore_barrier()

    chunks_per_tile = n_chunks_total // N_TILES
    n_iter = chunks_per_tile // 2
    c0 = tile * chunks_per_tile

    lanes_iota = lax.iota(jnp.int32, LANES)

    def _load_idx(chunk, raw, idx):
        pltpu.sync_copy(idx3_hbm.at[chunk], raw)
        for col in range(3):
            for k in range(CHUNK // LANES):
                pat = lanes_iota * 3 + (3 * LANES * k + col)
                idx[col, pl.ds(k * LANES, LANES)] = plsc.load_gather(
                    raw, [pat])

    _load_idx(c0, raw_a, idx_a)
    pltpu.async_copy(nodes_hbm.at[idx_a.at[0]], buf_a, sem_a)
    pltpu.async_copy(eemb_hbm.at[idx_a.at[1]], buf_b, sem_b)

    def body(i, carry):
        even = c0 + 2 * i
        _load_idx(even + 1, raw_b, idx_b)
        gc = pltpu.async_copy(nodes_hbm.at[idx_b.at[0]], buf_c, sem_c)
        gd = pltpu.async_copy(eemb_hbm.at[idx_b.at[1]], buf_d, sem_d)
        pltpu.make_async_copy(nodes_hbm.at[idx_a.at[0]], buf_a, sem_a).wait()
        pltpu.sync_copy(buf_a, acc_sum.at[idx_a.at[2]], add=True)
        pltpu.make_async_copy(eemb_hbm.at[idx_a.at[1]], buf_b, sem_b).wait()
        pltpu.sync_copy(buf_b, acc_sum.at[idx_a.at[2]], add=True)
        pltpu.sync_copy(ones_v, acc_cnt.at[idx_a.at[2]], add=True)
        @pl.when(i + 1 < n_iter)
        def _():
            _load_idx(even + 2, raw_a, idx_a)
            pltpu.async_copy(nodes_hbm.at[idx_a.at[0]], buf_a, sem_a)
            pltpu.async_copy(eemb_hbm.at[idx_a.at[1]], buf_b, sem_b)
        gc.wait()
        pltpu.sync_copy(buf_c, acc_sum.at[idx_b.at[2]], add=True)
        gd.wait()
        pltpu.sync_copy(buf_d, acc_sum.at[idx_b.at[2]], add=True)
        pltpu.sync_copy(ones_v, acc_cnt.at[idx_b.at[2]], add=True)
        return carry
    lax.fori_loop(0, n_iter, body, 0)

    plsc.subcore_barrier()
    for s in range(SUBCORES):
        @pl.when(sid == s)
        def _():
            r0 = s * rows_per_tile
            out_r0 = cid * n_pad + r0
            pltpu.sync_copy(acc_sum.at[pl.ds(r0, rows_per_tile)],
                            sum_out.at[pl.ds(out_r0, rows_per_tile)])
            pltpu.sync_copy(acc_cnt.at[pl.ds(r0, rows_per_tile)],
                            cnt_out.at[pl.ds(out_r0, rows_per_tile)])


def _sc_aggregate(idx3, nodes_embed, edges_embed, n_pad):
    d = nodes_embed.shape[1]
    rows_per_tile = n_pad // SUBCORES
    z_rows = jnp.zeros((rows_per_tile, d), jnp.float32)
    z_cnt = jnp.zeros((rows_per_tile, LANES), jnp.float32)
    ones = jnp.ones((CHUNK, LANES), jnp.float32)
    mesh = plsc.VectorSubcoreMesh(core_axis_name="c", subcore_axis_name="s")
    agg = pl.kernel(
        _sc_agg_body,
        out_type=(
            jax.ShapeDtypeStruct((2 * n_pad, d), jnp.float32),
            jax.ShapeDtypeStruct((2 * n_pad, LANES), jnp.float32),
        ),
        mesh=mesh,
        compiler_params=pltpu.CompilerParams(use_tc_tiling_on_sc=False,
                                             needs_layout_passes=False),
        scratch_types=[
            pltpu.VMEM((3 * CHUNK,), jnp.int32),
            pltpu.VMEM((3 * CHUNK,), jnp.int32),
            pltpu.VMEM((3, CHUNK), jnp.int32),
            pltpu.VMEM((3, CHUNK), jnp.int32),
            pltpu.VMEM((CHUNK, d), jnp.float32),
            pltpu.VMEM((CHUNK, d), jnp.float32),
            pltpu.VMEM((CHUNK, d), jnp.float32),
            pltpu.VMEM((CHUNK, d), jnp.float32),
            pltpu.VMEM((CHUNK, LANES), jnp.float32),
            pltpu.VMEM_SHARED((n_pad, d), jnp.float32),
            pltpu.VMEM_SHARED((n_pad, LANES), jnp.float32),
            pltpu.SemaphoreType.DMA,
            pltpu.SemaphoreType.DMA,
            pltpu.SemaphoreType.DMA,
            pltpu.SemaphoreType.DMA,
        ],
    )
    return agg(idx3, nodes_embed, edges_embed, z_rows, z_cnt, ones)


def _dense_body(nodes_ref, s0_ref, s1_ref, c0_ref, c1_ref, ws_ref, wa_ref,
                out_ref):
    x = nodes_ref[...]
    s = s0_ref[...] + s1_ref[...]
    c = c0_ref[...][:, 0:1] + c1_ref[...][:, 0:1]
    mean = s / jnp.maximum(c, 1.0)
    dims = (((1,), (1,)), ((), ()))
    h = lax.dot_general(x, ws_ref[...], dims,
                        preferred_element_type=jnp.float32)
    h = h + lax.dot_general(mean, wa_ref[...], dims,
                            preferred_element_type=jnp.float32)
    out_ref[...] = jnp.tanh(h)


def _dense(nodes_embed, s0, s1, c0, c1, W_self, W_agg):
    n, d = nodes_embed.shape
    blk = 1000
    grid = (n // blk,)
    row_spec = pl.BlockSpec((blk, d), lambda i: (i, 0))
    cnt_spec = pl.BlockSpec((blk, LANES), lambda i: (i, 0))
    w_spec = pl.BlockSpec((d, d), lambda i: (0, 0))
    return pl.pallas_call(
        _dense_body,
        grid=grid,
        in_specs=[row_spec, row_spec, row_spec, cnt_spec, cnt_spec, w_spec,
                  w_spec],
        out_specs=row_spec,
        out_shape=jax.ShapeDtypeStruct((n, d), jnp.float32),
    )(nodes_embed, s0, s1, c0, c1, W_self, W_agg)


def kernel(nodes_embed, edges_embed, edges, W_self, W_agg):
    n_nodes, d = nodes_embed.shape
    n_edges = edges.shape[0]
    per_tile = SUBCORES * 8
    n_pad = ((n_nodes + per_tile - 1) // per_tile) * per_tile
    unit = 2 * CHUNK * N_TILES
    n_edges_pad = ((n_edges + unit - 1) // unit) * unit
    pad_row = jnp.array([[0, 0, n_pad - 1]], jnp.int32)
    edges_p = jnp.concatenate(
        [edges, jnp.broadcast_to(pad_row, (n_edges_pad - n_edges, 3))], 0)
    idx3 = edges_p.reshape(-1, 3 * CHUNK)
    sums, cnts = _sc_aggregate(idx3, nodes_embed, edges_embed, n_pad)
    out = _dense(nodes_embed, sums[:n_nodes], sums[n_pad:n_pad + n_nodes],
                 cnts[:n_nodes], cnts[n_pad:n_pad + n_nodes], W_self, W_agg)
    return out


# speedup vs baseline: 1.2751x; 1.2751x over previous
import functools

import jax
import jax.numpy as jnp
from jax import lax
from jax.experimental import pallas as pl
from jax.experimental.pallas import tpu as pltpu
from jax.experimental.pallas import tpu_sc as plsc

N_TILES = 32
SUBCORES = 16
CHUNK = 64
LANES = 16


def _sc_agg_body(idx3_hbm, nodes_hbm, eemb_hbm, z_rows_hbm,
                 z_cnt_hbm, ones_hbm, sum_out, cnt_out,
                 idx_a, idx_b, buf_a, buf_b, buf_c, buf_d,
                 ones_v, acc_sum, acc_cnt, sem_a, sem_b, sem_c, sem_d):
    n_chunks_total = idx3_hbm.shape[0]
    n_pad, d = acc_sum.shape
    rows_per_tile = n_pad // SUBCORES
    cid = lax.axis_index("c")
    sid = lax.axis_index("s")
    tile = cid * SUBCORES + sid

    pltpu.sync_copy(ones_hbm, ones_v)
    for s in range(SUBCORES):
        @pl.when(sid == s)
        def _():
            pltpu.sync_copy(z_rows_hbm, acc_sum.at[pl.ds(s * rows_per_tile,
                                                         rows_per_tile)])
            pltpu.sync_copy(z_cnt_hbm, acc_cnt.at[pl.ds(s * rows_per_tile,
                                                        rows_per_tile)])
    plsc.subcore_barrier()

    chunks_per_tile = n_chunks_total // N_TILES
    n_iter = chunks_per_tile // 2
    c0 = tile * chunks_per_tile

    def _load_idx(chunk, idx):
        pltpu.sync_copy(idx3_hbm.at[chunk], idx)

    _load_idx(c0, idx_a)
    pltpu.async_copy(nodes_hbm.at[idx_a.at[0]], buf_a, sem_a)
    pltpu.async_copy(eemb_hbm.at[idx_a.at[1]], buf_b, sem_b)

    def body(i, carry):
        even = c0 + 2 * i
        _load_idx(even + 1, idx_b)
        gc = pltpu.async_copy(nodes_hbm.at[idx_b.at[0]], buf_c, sem_c)
        gd = pltpu.async_copy(eemb_hbm.at[idx_b.at[1]], buf_d, sem_d)
        pltpu.make_async_copy(nodes_hbm.at[idx_a.at[0]], buf_a, sem_a).wait()
        pltpu.sync_copy(buf_a, acc_sum.at[idx_a.at[2]], add=True)
        pltpu.make_async_copy(eemb_hbm.at[idx_a.at[1]], buf_b, sem_b).wait()
        pltpu.sync_copy(buf_b, acc_sum.at[idx_a.at[2]], add=True)
        pltpu.sync_copy(ones_v, acc_cnt.at[idx_a.at[2]], add=True)
        @pl.when(i + 1 < n_iter)
        def _():
            _load_idx(even + 2, idx_a)
            pltpu.async_copy(nodes_hbm.at[idx_a.at[0]], buf_a, sem_a)
            pltpu.async_copy(eemb_hbm.at[idx_a.at[1]], buf_b, sem_b)
        gc.wait()
        pltpu.sync_copy(buf_c, acc_sum.at[idx_b.at[2]], add=True)
        gd.wait()
        pltpu.sync_copy(buf_d, acc_sum.at[idx_b.at[2]], add=True)
        pltpu.sync_copy(ones_v, acc_cnt.at[idx_b.at[2]], add=True)
        return carry
    lax.fori_loop(0, n_iter, body, 0)

    plsc.subcore_barrier()
    for s in range(SUBCORES):
        @pl.when(sid == s)
        def _():
            r0 = s * rows_per_tile
            out_r0 = cid * n_pad + r0
            pltpu.sync_copy(acc_sum.at[pl.ds(r0, rows_per_tile)],
                            sum_out.at[pl.ds(out_r0, rows_per_tile)])
            pltpu.sync_copy(acc_cnt.at[pl.ds(r0, rows_per_tile)],
                            cnt_out.at[pl.ds(out_r0, rows_per_tile)])


def _sc_aggregate(idx3, nodes_embed, edges_embed, n_pad):
    d = nodes_embed.shape[1]
    rows_per_tile = n_pad // SUBCORES
    z_rows = jnp.zeros((rows_per_tile, d), jnp.float32)
    z_cnt = jnp.zeros((rows_per_tile, LANES), jnp.float32)
    ones = jnp.ones((CHUNK, LANES), jnp.float32)
    mesh = plsc.VectorSubcoreMesh(core_axis_name="c", subcore_axis_name="s")
    agg = pl.kernel(
        _sc_agg_body,
        out_type=(
            jax.ShapeDtypeStruct((2 * n_pad, d), jnp.float32),
            jax.ShapeDtypeStruct((2 * n_pad, LANES), jnp.float32),
        ),
        mesh=mesh,
        compiler_params=pltpu.CompilerParams(use_tc_tiling_on_sc=False),
        scratch_types=[
            pltpu.VMEM((3, CHUNK), jnp.int32),
            pltpu.VMEM((3, CHUNK), jnp.int32),
            pltpu.VMEM((CHUNK, d), jnp.float32),
            pltpu.VMEM((CHUNK, d), jnp.float32),
            pltpu.VMEM((CHUNK, d), jnp.float32),
            pltpu.VMEM((CHUNK, d), jnp.float32),
            pltpu.VMEM((CHUNK, LANES), jnp.float32),
            pltpu.VMEM_SHARED((n_pad, d), jnp.float32),
            pltpu.VMEM_SHARED((n_pad, LANES), jnp.float32),
            pltpu.SemaphoreType.DMA,
            pltpu.SemaphoreType.DMA,
            pltpu.SemaphoreType.DMA,
            pltpu.SemaphoreType.DMA,
        ],
    )
    return agg(idx3, nodes_embed, edges_embed, z_rows, z_cnt, ones)


def _dense_body(nodes_ref, s0_ref, s1_ref, c0_ref, c1_ref, ws_ref, wa_ref,
                out_ref):
    x = nodes_ref[...]
    s = s0_ref[...] + s1_ref[...]
    c = c0_ref[...][:, 0:1] + c1_ref[...][:, 0:1]
    mean = s / jnp.maximum(c, 1.0)
    dims = (((1,), (1,)), ((), ()))
    h = lax.dot_general(x, ws_ref[...], dims,
                        preferred_element_type=jnp.float32)
    h = h + lax.dot_general(mean, wa_ref[...], dims,
                            preferred_element_type=jnp.float32)
    out_ref[...] = jnp.tanh(h)


def _dense(nodes_embed, s0, s1, c0, c1, W_self, W_agg):
    n, d = nodes_embed.shape
    blk = 1000
    grid = (n // blk,)
    row_spec = pl.BlockSpec((blk, d), lambda i: (i, 0))
    cnt_spec = pl.BlockSpec((blk, LANES), lambda i: (i, 0))
    w_spec = pl.BlockSpec((d, d), lambda i: (0, 0))
    return pl.pallas_call(
        _dense_body,
        grid=grid,
        in_specs=[row_spec, row_spec, row_spec, cnt_spec, cnt_spec, w_spec,
                  w_spec],
        out_specs=row_spec,
        out_shape=jax.ShapeDtypeStruct((n, d), jnp.float32),
    )(nodes_embed, s0, s1, c0, c1, W_self, W_agg)


def kernel(nodes_embed, edges_embed, edges, W_self, W_agg):
    n_nodes, d = nodes_embed.shape
    n_edges = edges.shape[0]
    per_tile = SUBCORES * 8
    n_pad = ((n_nodes + per_tile - 1) // per_tile) * per_tile
    unit = 2 * CHUNK * N_TILES
    n_edges_pad = ((n_edges + unit - 1) // unit) * unit
    pad_row = jnp.array([[0, 0, n_pad - 1]], jnp.int32)
    edges_p = jnp.concatenate(
        [edges, jnp.broadcast_to(pad_row, (n_edges_pad - n_edges, 3))], 0)
    idx3 = edges_p.reshape(-1, CHUNK, 3).transpose(0, 2, 1)
    sums, cnts = _sc_aggregate(idx3, nodes_embed, edges_embed, n_pad)
    out = _dense(nodes_embed, sums[:n_nodes], sums[n_pad:n_pad + n_nodes],
                 cnts[:n_nodes], cnts[n_pad:n_pad + n_nodes], W_self, W_agg)
    return out


# speedup vs baseline: 1.4820x; 1.1623x over previous
import functools

import jax
import jax.numpy as jnp
from jax import lax
from jax.experimental import pallas as pl
from jax.experimental.pallas import tpu as pltpu
from jax.experimental.pallas import tpu_sc as plsc

N_TILES = 32
SUBCORES = 16
CHUNK = 40
LANES = 16


def _sc_agg_body(idx3_hbm, nodes_hbm, eemb_hbm, z_rows_hbm,
                 z_cnt_hbm, ones_hbm, sum_out, cnt_out,
                 idx_a, idx_b, buf_a, buf_b, buf_c, buf_d,
                 ones_v, acc_sum, acc_cnt, sem_a, sem_b, sem_c, sem_d):
    n_chunks_total = idx3_hbm.shape[0]
    n_pad, d = acc_sum.shape
    rows_per_tile = n_pad // SUBCORES
    cid = lax.axis_index("c")
    sid = lax.axis_index("s")
    tile = cid * SUBCORES + sid

    pltpu.sync_copy(ones_hbm, ones_v)
    for s in range(SUBCORES):
        @pl.when(sid == s)
        def _():
            pltpu.sync_copy(z_rows_hbm, acc_sum.at[pl.ds(s * rows_per_tile,
                                                         rows_per_tile)])
            pltpu.sync_copy(z_cnt_hbm, acc_cnt.at[pl.ds(s * rows_per_tile,
                                                        rows_per_tile)])
    plsc.subcore_barrier()

    chunks_per_tile = n_chunks_total // N_TILES
    n_iter = chunks_per_tile // 2
    c0 = tile * chunks_per_tile

    def _load_idx(chunk, idx):
        pltpu.sync_copy(idx3_hbm.at[chunk], idx)

    _load_idx(c0, idx_a)
    pltpu.async_copy(nodes_hbm.at[idx_a.at[0]], buf_a, sem_a)
    pltpu.async_copy(eemb_hbm.at[idx_a.at[1]], buf_b, sem_b)

    def body(i, carry):
        even = c0 + 2 * i
        _load_idx(even + 1, idx_b)
        gc = pltpu.async_copy(nodes_hbm.at[idx_b.at[0]], buf_c, sem_c)
        gd = pltpu.async_copy(eemb_hbm.at[idx_b.at[1]], buf_d, sem_d)
        pltpu.make_async_copy(nodes_hbm.at[idx_a.at[0]], buf_a, sem_a).wait()
        pltpu.sync_copy(buf_a, acc_sum.at[idx_a.at[2]], add=True)
        pltpu.make_async_copy(eemb_hbm.at[idx_a.at[1]], buf_b, sem_b).wait()
        pltpu.sync_copy(buf_b, acc_sum.at[idx_a.at[2]], add=True)
        pltpu.sync_copy(ones_v, acc_cnt.at[idx_a.at[2]], add=True)
        @pl.when(i + 1 < n_iter)
        def _():
            _load_idx(even + 2, idx_a)
            pltpu.async_copy(nodes_hbm.at[idx_a.at[0]], buf_a, sem_a)
            pltpu.async_copy(eemb_hbm.at[idx_a.at[1]], buf_b, sem_b)
        gc.wait()
        pltpu.sync_copy(buf_c, acc_sum.at[idx_b.at[2]], add=True)
        gd.wait()
        pltpu.sync_copy(buf_d, acc_sum.at[idx_b.at[2]], add=True)
        pltpu.sync_copy(ones_v, acc_cnt.at[idx_b.at[2]], add=True)
        return carry
    lax.fori_loop(0, n_iter, body, 0)

    plsc.subcore_barrier()
    for s in range(SUBCORES):
        @pl.when(sid == s)
        def _():
            r0 = s * rows_per_tile
            out_r0 = cid * n_pad + r0
            pltpu.sync_copy(acc_sum.at[pl.ds(r0, rows_per_tile)],
                            sum_out.at[pl.ds(out_r0, rows_per_tile)])
            pltpu.sync_copy(acc_cnt.at[pl.ds(r0, rows_per_tile)],
                            cnt_out.at[pl.ds(out_r0, rows_per_tile)])


def _sc_aggregate(idx3, nodes_embed, edges_embed, n_pad):
    d = nodes_embed.shape[1]
    rows_per_tile = n_pad // SUBCORES
    z_rows = jnp.zeros((rows_per_tile, d), jnp.float32)
    z_cnt = jnp.zeros((rows_per_tile, LANES), jnp.float32)
    ones = jnp.ones((CHUNK, LANES), jnp.float32)
    mesh = plsc.VectorSubcoreMesh(core_axis_name="c", subcore_axis_name="s")
    agg = pl.kernel(
        _sc_agg_body,
        out_type=(
            jax.ShapeDtypeStruct((2 * n_pad, d), jnp.float32),
            jax.ShapeDtypeStruct((2 * n_pad, LANES), jnp.float32),
        ),
        mesh=mesh,
        compiler_params=pltpu.CompilerParams(use_tc_tiling_on_sc=False),
        scratch_types=[
            pltpu.VMEM((3, CHUNK), jnp.int32),
            pltpu.VMEM((3, CHUNK), jnp.int32),
            pltpu.VMEM((CHUNK, d), jnp.float32),
            pltpu.VMEM((CHUNK, d), jnp.float32),
            pltpu.VMEM((CHUNK, d), jnp.float32),
            pltpu.VMEM((CHUNK, d), jnp.float32),
            pltpu.VMEM((CHUNK, LANES), jnp.float32),
            pltpu.VMEM_SHARED((n_pad, d), jnp.float32),
            pltpu.VMEM_SHARED((n_pad, LANES), jnp.float32),
            pltpu.SemaphoreType.DMA,
            pltpu.SemaphoreType.DMA,
            pltpu.SemaphoreType.DMA,
            pltpu.SemaphoreType.DMA,
        ],
    )
    return agg(idx3, nodes_embed, edges_embed, z_rows, z_cnt, ones)


def _dense_body(nodes_ref, s0_ref, s1_ref, c0_ref, c1_ref, ws_ref, wa_ref,
                out_ref):
    x = nodes_ref[...]
    s = s0_ref[...] + s1_ref[...]
    c = c0_ref[...][:, 0:1] + c1_ref[...][:, 0:1]
    mean = s / jnp.maximum(c, 1.0)
    dims = (((1,), (1,)), ((), ()))
    h = lax.dot_general(x, ws_ref[...], dims,
                        preferred_element_type=jnp.float32)
    h = h + lax.dot_general(mean, wa_ref[...], dims,
                            preferred_element_type=jnp.float32)
    out_ref[...] = jnp.tanh(h)


def _dense(nodes_embed, s0, s1, c0, c1, W_self, W_agg):
    n, d = nodes_embed.shape
    blk = 1000
    grid = (n // blk,)
    row_spec = pl.BlockSpec((blk, d), lambda i: (i, 0))
    cnt_spec = pl.BlockSpec((blk, LANES), lambda i: (i, 0))
    w_spec = pl.BlockSpec((d, d), lambda i: (0, 0))
    return pl.pallas_call(
        _dense_body,
        grid=grid,
        in_specs=[row_spec, row_spec, row_spec, cnt_spec, cnt_spec, w_spec,
                  w_spec],
        out_specs=row_spec,
        out_shape=jax.ShapeDtypeStruct((n, d), jnp.float32),
    )(nodes_embed, s0, s1, c0, c1, W_self, W_agg)


def kernel(nodes_embed, edges_embed, edges, W_self, W_agg):
    n_nodes, d = nodes_embed.shape
    n_edges = edges.shape[0]
    per_tile = SUBCORES * 8
    n_pad = ((n_nodes + per_tile - 1) // per_tile) * per_tile
    unit = 2 * CHUNK * N_TILES
    n_edges_pad = ((n_edges + unit - 1) // unit) * unit
    pad_row = jnp.array([[0, 0, n_pad - 1]], jnp.int32)
    edges_p = jnp.concatenate(
        [edges, jnp.broadcast_to(pad_row, (n_edges_pad - n_edges, 3))], 0)
    idx3 = edges_p.reshape(-1, CHUNK, 3).transpose(0, 2, 1)
    sums, cnts = _sc_aggregate(idx3, nodes_embed, edges_embed, n_pad)
    out = _dense(nodes_embed, sums[:n_nodes], sums[n_pad:n_pad + n_nodes],
                 cnts[:n_nodes], cnts[n_pad:n_pad + n_nodes], W_self, W_agg)
    return out
